# baseline (speedup 1.0000x reference)
import functools

import jax
import jax.numpy as jnp
from jax import lax
from jax.experimental import pallas as pl
from jax.experimental.pallas import tpu as pltpu
from jax.experimental.pallas import tpu_sc as plsc

H = 128
NV = H // 16
EPS = 1e-12
MAGIC = 0x5F3759DF
CSZ = 128
U = 16


def _rsqrt_vec(v):
    i = lax.bitcast_convert_type(v, jnp.int32)
    y = lax.bitcast_convert_type(
        jnp.full((16,), MAGIC, jnp.int32) - (i >> 1), jnp.float32)
    half = v * 0.5
    for _ in range(2):
        y = y * (1.5 - half * y * y)
    return y


def _sc_embed_ln(ids_flat, word_table, pos_ext, gamma, beta, *, n_tok, seq):
    info = plsc.get_sparse_core_info()
    nw = info.num_cores * info.num_subcores
    tok_per_w = n_tok // nw
    n_chunks = tok_per_w // CSZ
    pos_rows = pos_ext.shape[0]
    assert n_chunks * CSZ == tok_per_w and n_chunks >= 2

    mesh = plsc.VectorSubcoreMesh(core_axis_name="c", subcore_axis_name="s")

    @functools.partial(
        pl.kernel,
        out_type=jax.ShapeDtypeStruct((n_tok, H), jnp.float32),
        mesh=mesh,
        compiler_params=pltpu.CompilerParams(needs_layout_passes=False),
        scratch_types=[
            pltpu.VMEM((tok_per_w,), jnp.int32),
            pltpu.VMEM((pos_rows, H), jnp.float32),
            pltpu.VMEM((2, CSZ, H), jnp.float32),
            pltpu.VMEM((2, CSZ, H), jnp.float32),
            pltpu.VMEM((H,), jnp.float32),
            pltpu.VMEM((H,), jnp.float32),
            pltpu.SemaphoreType.DMA((2,)),
            pltpu.SemaphoreType.DMA((2,)),
        ],
    )
    def k(ids_hbm, word_hbm, pos_hbm, gamma_hbm, beta_hbm, out_hbm,
          idx_v, pos_v, irows, orows, gamma_v, beta_v, gsem, osem):
        cid = lax.axis_index("c")
        sid = lax.axis_index("s")
        wid = sid * info.num_cores + cid
        base = wid * tok_per_w

        pltpu.sync_copy(ids_hbm.at[pl.ds(base, tok_per_w)], idx_v)
        pltpu.sync_copy(pos_hbm, pos_v)
        pltpu.sync_copy(gamma_hbm, gamma_v)
        pltpu.sync_copy(beta_hbm, beta_v)
        g = [gamma_v[pl.ds(16 * j, 16)] for j in range(NV)]
        bt = [beta_v[pl.ds(16 * j, 16)] for j in range(NV)]

        def gather(c, b):
            off = pl.multiple_of(c * CSZ, 8)
            return pltpu.async_copy(
                word_hbm.at[idx_v.at[pl.ds(off, CSZ)]], irows.at[b],
                gsem.at[b])

        def wb_copy(c, b):
            off = pl.multiple_of(c * CSZ, 8)
            return pltpu.make_async_copy(
                orows.at[b], out_hbm.at[pl.ds(base + off, CSZ)], osem.at[b])

        gather(0, 0)

        def _tree_sum(vals):
            vals = list(vals)
            while len(vals) > 1:
                vals = [a + b for a, b in zip(vals[::2], vals[1::2])]
            return vals[0]

        gb_dev = _tree_sum([jnp.abs(g[j] - 1.0) for j in range(NV)]
                           + [jnp.abs(bt[j]) for j in range(NV)])
        plain_affine = jnp.sum(gb_dev) == 0.0

        def _run_tokens(affine, bi, bo, poff):
            @plsc.parallel_loop(0, CSZ, unroll=U, carry=jnp.int32(0))
            def _loop(t, cc):
                x = []
                for j in range(NV):
                    sl = pl.ds(16 * j, 16)
                    x.append(irows[bi, t, sl] + pos_v[poff + t, sl])
                tot = jnp.sum(_tree_sum(x))
                tot2 = jnp.sum(_tree_sum([v * v for v in x]))
                mean = tot * (1.0 / H)
                var = tot2 * (1.0 / H) - mean * mean
                rstd = _rsqrt_vec(jnp.full((16,), var + EPS, jnp.float32))
                mean_v = jnp.full((16,), mean, jnp.float32)
                for j in range(NV):
                    sl = pl.ds(16 * j, 16)
                    y = (x[j] - mean_v) * rstd
                    orows[bo, t, sl] = y * g[j] + bt[j] if affine else y
                return cc

        def chunk_body(c, _):
            bi = lax.rem(c, 2)
            bo = lax.rem(c, 2)
            off = pl.multiple_of(c * CSZ, 8)
            pltpu.make_async_copy(
                word_hbm.at[idx_v.at[pl.ds(off, CSZ)]], irows.at[bi],
                gsem.at[bi]).wait()

            @pl.when(c + 1 < n_chunks)
            def _prefetch():
                gather(c + 1, lax.rem(c + 1, 2))

            @pl.when(c >= 2)
            def _drain_wb():
                wb_copy(c - 2, bo).wait()

            poff = lax.rem(c * CSZ, seq)

            @pl.when(plain_affine)
            def _plain():
                _run_tokens(False, bi, bo, poff)

            @pl.when(jnp.logical_not(plain_affine))
            def _affine():
                _run_tokens(True, bi, bo, poff)

            pltpu.async_copy(
                orows.at[bo], out_hbm.at[pl.ds(base + off, CSZ)], osem.at[bo])
            return 0

        lax.fori_loop(0, n_chunks, chunk_body, 0)

        for c in range(n_chunks - 2, n_chunks):
            wb_copy(c, c % 2).wait()

    return k(ids_flat, word_table, pos_ext, gamma, beta)


def kernel(input_ids, word_table, pos_table, gamma, beta):
    b, s = input_ids.shape
    n_tok = b * s
    ids_flat = input_ids.reshape(n_tok).astype(jnp.int32)
    pos_s = pos_table[:s]
    pos_ext = jnp.concatenate([pos_s, pos_s[:320 - s]], axis=0)
    out = _sc_embed_ln(ids_flat, word_table, pos_ext, gamma, beta,
                       n_tok=n_tok, seq=s)
    return out.reshape(b, s, H)


# speedup vs baseline: 1.0639x; 1.0639x over previous
import functools

import jax
import jax.numpy as jnp
from jax import lax
from jax.experimental import pallas as pl
from jax.experimental.pallas import tpu as pltpu
from jax.experimental.pallas import tpu_sc as plsc

H = 128
NV = H // 16
EPS = 1e-12
MAGIC = 0x5F3759DF
NBUF = 3


def _rsqrt_vec(v):
    i = lax.bitcast_convert_type(v, jnp.int32)
    y = lax.bitcast_convert_type(
        jnp.full((16,), MAGIC, jnp.int32) - (i >> 1), jnp.float32)
    half = v * 0.5
    for _ in range(2):
        y = y * (1.5 - half * y * y)
    return y


def _sc_embed_ln(ids_flat, word_table, pos_table, gamma, beta, *, n_tok, seq):
    info = plsc.get_sparse_core_info()
    nw = info.num_cores * info.num_subcores
    tok_per_w = n_tok // nw
    n_chunks = tok_per_w // seq
    assert n_chunks >= NBUF

    mesh = plsc.VectorSubcoreMesh(core_axis_name="c", subcore_axis_name="s")

    @functools.partial(
        pl.kernel,
        out_type=jax.ShapeDtypeStruct((n_tok, H), jnp.float32),
        mesh=mesh,
        compiler_params=pltpu.CompilerParams(needs_layout_passes=False),
        scratch_types=[
            pltpu.VMEM((tok_per_w,), jnp.int32),
            pltpu.VMEM((seq, H), jnp.float32),
            pltpu.VMEM((NBUF, seq, H), jnp.float32),
            pltpu.VMEM((H,), jnp.float32),
            pltpu.VMEM((H,), jnp.float32),
            pltpu.SemaphoreType.DMA((NBUF,)),
            pltpu.SemaphoreType.DMA((NBUF,)),
        ],
    )
    def k(ids_hbm, word_hbm, pos_hbm, gamma_hbm, beta_hbm, out_hbm,
          idx_v, pos_v, rows, gamma_v, beta_v, gsem, osem):
        cid = lax.axis_index("c")
        sid = lax.axis_index("s")
        wid = sid * info.num_cores + cid
        base = wid * tok_per_w

        pltpu.sync_copy(ids_hbm.at[pl.ds(base, tok_per_w)], idx_v)
        pltpu.sync_copy(pos_hbm.at[pl.ds(0, seq)], pos_v)
        pltpu.sync_copy(gamma_hbm, gamma_v)
        pltpu.sync_copy(beta_hbm, beta_v)
        g = [gamma_v[pl.ds(16 * j, 16)] for j in range(NV)]
        bt = [beta_v[pl.ds(16 * j, 16)] for j in range(NV)]

        def gather(c, b):
            off = pl.multiple_of(c * seq, 8)
            return pltpu.async_copy(
                word_hbm.at[idx_v.at[pl.ds(off, seq)]], rows.at[b],
                gsem.at[b])

        def wb_copy(c, b):
            off = pl.multiple_of(c * seq, 8)
            return pltpu.make_async_copy(
                rows.at[b], out_hbm.at[pl.ds(base + off, seq)], osem.at[b])

        for c in range(NBUF - 1):
            gather(c, c)

        def _tree_sum(vals):
            vals = list(vals)
            while len(vals) > 1:
                vals = [a + b for a, b in zip(vals[::2], vals[1::2])]
            return vals[0]

        U = 25

        gb_dev = _tree_sum([jnp.abs(g[j] - 1.0) for j in range(NV)]
                           + [jnp.abs(bt[j]) for j in range(NV)])
        plain_affine = jnp.sum(gb_dev) == 0.0

        def _run_tokens(affine, b):
            @plsc.parallel_loop(0, seq, unroll=U, carry=jnp.int32(0))
            def _loop(t, cc):
                x = []
                for j in range(NV):
                    sl = pl.ds(16 * j, 16)
                    x.append(rows[b, t, sl] + pos_v[t, sl])
                tot = jnp.sum(_tree_sum(x))
                tot2 = jnp.sum(_tree_sum([v * v for v in x]))
                mean = tot * (1.0 / H)
                var = tot2 * (1.0 / H) - mean * mean
                rstd = _rsqrt_vec(jnp.full((16,), var + EPS, jnp.float32))
                mean_v = jnp.full((16,), mean, jnp.float32)
                for j in range(NV):
                    sl = pl.ds(16 * j, 16)
                    y = (x[j] - mean_v) * rstd
                    rows[b, t, sl] = y * g[j] + bt[j] if affine else y
                return cc

        def chunk_body(c, _):
            b = lax.rem(c, NBUF)
            off = pl.multiple_of(c * seq, 8)
            pltpu.make_async_copy(
                word_hbm.at[idx_v.at[pl.ds(off, seq)]], rows.at[b],
                gsem.at[b]).wait()

            @pl.when(plain_affine)
            def _plain():
                _run_tokens(False, b)

            @pl.when(jnp.logical_not(plain_affine))
            def _affine():
                _run_tokens(True, b)

            pltpu.async_copy(
                rows.at[b], out_hbm.at[pl.ds(base + off, seq)], osem.at[b])
            cn = c + (NBUF - 1)

            @pl.when(cn < n_chunks)
            def _prefetch():
                bn = lax.rem(cn, NBUF)

                @pl.when(c >= 1)
                def _drain_prev():
                    wb_copy(c - 1, bn).wait()

                gather(cn, bn)

            return 0

        lax.fori_loop(0, n_chunks, chunk_body, 0)

        for c in range(n_chunks - NBUF, n_chunks):
            wb_copy(c, c % NBUF).wait()

    return k(ids_flat, word_table, pos_table, gamma, beta)


def kernel(input_ids, word_table, pos_table, gamma, beta):
    b, s = input_ids.shape
    n_tok = b * s
    ids_flat = input_ids.reshape(n_tok).astype(jnp.int32)
    out = _sc_embed_ln(ids_flat, word_table, pos_table, gamma, beta,
                       n_tok=n_tok, seq=s)
    return out.reshape(b, s, H)
